# baseline (speedup 1.0000x reference)
import dataclasses
import functools

import numpy as np

import jax
import jax.numpy as jnp
from jax import lax
from jax.experimental import pallas as pl
from jax.experimental.pallas import tpu as pltpu
from jax.experimental.pallas import tpu_sc as plsc

N = 524288
K = 512
NC = 2
NS = 16
NW = NC * NS
L = 16
CHUNK = N // NW
T_LEN = 2 * K
REG_LEVELS = 2
GATHER_LEVELS = 9 - REG_LEVELS
EPS = K // NS


def _tree_index_maps():
    perm = np.zeros(K - 1, np.int32)
    for j in range(K - 1):
        lvl = (j + 1).bit_length() - 1
        pos = j - ((1 << lvl) - 1)
        span = K >> lvl
        perm[j] = pos * span + span // 2 - 1
    a = np.zeros(T_LEN, np.int32)
    b = np.zeros(T_LEN, np.int32)
    a[: K - 1] = perm
    b[: K - 1] = perm + 1
    a[K - 1 : T_LEN - 1] = b[K - 1 : T_LEN - 1] = np.arange(K)
    return a, b


_A_MAP, _B_MAP = _tree_index_maps()


def _vq_body(param_hbm, cb_hbm, a_hbm, b_hbm, out_hbm,
             t_v, x_v, o_v, cb_v, s_v, a_v, b_v, rank_v, val_v,
             shared_s, sem):
    cid = lax.axis_index("c")
    sid = lax.axis_index("s")
    wid = sid * NC + cid
    base = wid * CHUNK

    x_dma = pltpu.async_copy(param_hbm.at[pl.ds(base, CHUNK)], x_v, sem)

    pltpu.sync_copy(cb_hbm, cb_v)
    pltpu.sync_copy(a_hbm, a_v)
    pltpu.sync_copy(b_hbm, b_v)

    ebase = sid * EPS
    lane = lax.iota(jnp.int32, L)
    my = [cb_v[pl.ds(ebase + u * L, L)] for u in range(EPS // L)]
    idx = [lane + (ebase + u * L) for u in range(EPS // L)]

    def _rank_blk(blk, rk):
        w = cb_v[pl.ds(blk * L, L)]
        rk = list(rk)
        for t in range(L):
            s = jnp.full((L,), w[t])
            jj = blk * L + t
            for u in range(EPS // L):
                less = (s < my[u]) | ((s == my[u]) & (jj < idx[u]))
                rk[u] = jnp.where(less, rk[u] + 1, rk[u])
        return tuple(rk)

    rk = lax.fori_loop(
        0, K // L, _rank_blk,
        tuple(jnp.zeros((L,), jnp.int32) for _ in range(EPS // L)),
    )
    for u in range(EPS // L):
        rank_v[pl.ds(u * L, L)] = rk[u]
        val_v[pl.ds(u * L, L)] = my[u]
    pltpu.sync_copy(val_v, shared_s.at[rank_v])
    plsc.subcore_barrier()
    pltpu.sync_copy(shared_s, s_v)

    @pl.loop(0, T_LEN, step=L)
    def _(i):
        av = a_v[pl.ds(i, L)]
        bv = b_v[pl.ds(i, L)]
        ga = plsc.load_gather(s_v, [av])
        gb = plsc.load_gather(s_v, [bv])
        t_v[pl.ds(i, L)] = 0.5 * (ga + gb)

    x_dma.wait()

    sp = t_v[pl.ds(0, L)]
    splits = [jnp.full((L,), sp[t]) for t in range((1 << REG_LEVELS) - 1)]

    @plsc.parallel_loop(0, CHUNK, step=L, unroll=8)
    def _(i):
        x = x_v[pl.ds(i, L)]
        j = jnp.full((L,), (1 << REG_LEVELS) - 1, jnp.int32)
        for s in splits:
            j = jnp.where(s <= x, j + 1, j)
        for _lvl in range(GATHER_LEVELS):
            m = plsc.load_gather(t_v, [j])
            j = j + j + jnp.where(m <= x, 2, 1)
        o_v[pl.ds(i, L)] = plsc.load_gather(t_v, [j])

    pltpu.sync_copy(o_v, out_hbm.at[pl.ds(base, CHUNK)])


@functools.cache
def _make_vq():
    cp = pltpu.CompilerParams()
    if "needs_layout_passes" in pltpu.CompilerParams.__dataclass_fields__:
        cp = dataclasses.replace(cp, needs_layout_passes=False)
    return pl.kernel(
        _vq_body,
        compiler_params=cp,
        out_type=jax.ShapeDtypeStruct((N,), jnp.float32),
        mesh=plsc.VectorSubcoreMesh(core_axis_name="c", subcore_axis_name="s"),
        scratch_types=[
            pltpu.VMEM((T_LEN,), jnp.float32),
            pltpu.VMEM((CHUNK,), jnp.float32),
            pltpu.VMEM((CHUNK,), jnp.float32),
            pltpu.VMEM((K,), jnp.float32),
            pltpu.VMEM((K,), jnp.float32),
            pltpu.VMEM((T_LEN,), jnp.int32),
            pltpu.VMEM((T_LEN,), jnp.int32),
            pltpu.VMEM((EPS,), jnp.int32),
            pltpu.VMEM((EPS,), jnp.float32),
            pltpu.VMEM_SHARED((K,), jnp.float32),
            pltpu.SemaphoreType.DMA,
        ],
    )


@jax.jit
def kernel(param, y, codebook):
    quantized = _make_vq()(
        param,
        codebook.reshape(K),
        jnp.asarray(_A_MAP),
        jnp.asarray(_B_MAP),
    )
    return (quantized, y)


# speedup vs baseline: 1.4949x; 1.4949x over previous
import dataclasses
import functools

import numpy as np

import jax
import jax.numpy as jnp
from jax import lax
from jax.experimental import pallas as pl
from jax.experimental.pallas import tpu as pltpu
from jax.experimental.pallas import tpu_sc as plsc

N = 524288
K = 512
NC = 2
NS = 16
NW = NC * NS
L = 16
CHUNK = N // NW
T_LEN = 2 * K
REG_LEVELS = 2
GATHER_LEVELS = 9 - REG_LEVELS


def _tree_index_maps():
    perm = np.zeros(K - 1, np.int32)
    for j in range(K - 1):
        lvl = (j + 1).bit_length() - 1
        pos = j - ((1 << lvl) - 1)
        span = K >> lvl
        perm[j] = pos * span + span // 2 - 1
    a = np.zeros(T_LEN, np.int32)
    b = np.zeros(T_LEN, np.int32)
    a[: K - 1] = perm
    b[: K - 1] = perm + 1
    a[K - 1 : T_LEN - 1] = b[K - 1 : T_LEN - 1] = np.arange(K)
    return a.reshape(1, T_LEN), b.reshape(1, T_LEN)


_A_MAP, _B_MAP = _tree_index_maps()


def _prep_body(cb_col_ref, cb_row_ref, a_ref, b_ref, t_ref):
    ci = cb_col_ref[...]
    cj = cb_row_ref[...]
    ii = lax.broadcasted_iota(jnp.int32, (K, K), 0)
    jj = lax.broadcasted_iota(jnp.int32, (K, K), 1)
    gt = (ci > cj) | ((ci == cj) & (ii > jj))
    rank = jnp.sum(gt.astype(jnp.int32), axis=1, keepdims=True)
    sel = (rank == a_ref[...]).astype(jnp.float32) + (
        rank == b_ref[...]
    ).astype(jnp.float32)
    t_ref[...] = 0.5 * jnp.sum(sel * ci, axis=0, keepdims=True)


_prep = pl.pallas_call(
    _prep_body,
    out_shape=jax.ShapeDtypeStruct((1, T_LEN), jnp.float32),
)


NQ = 4
QC = CHUNK // NQ


def _vq_body(param_hbm, t_hbm, out_hbm, t_v, x_v, o_v,
             in_s0, in_s1, in_s2, in_s3, out_sem):
    wid = lax.axis_index("s") * NC + lax.axis_index("c")
    base = wid * CHUNK
    in_sems = (in_s0, in_s1, in_s2, in_s3)

    in_copies = [
        pltpu.async_copy(
            param_hbm.at[pl.ds(base + q * QC, QC)],
            x_v.at[pl.ds(q * QC, QC)],
            in_sems[q],
        )
        for q in range(NQ)
    ]
    pltpu.sync_copy(t_hbm, t_v)

    sp = t_v[pl.ds(0, L)]
    splits = [jnp.full((L,), sp[t]) for t in range((1 << REG_LEVELS) - 1)]

    out_copies = []
    for q in range(NQ):
        in_copies[q].wait()

        @plsc.parallel_loop(q * QC, (q + 1) * QC, step=L, unroll=4)
        def _(i):
            x = x_v[pl.ds(i, L)]
            j = jnp.full((L,), (1 << REG_LEVELS) - 1, jnp.int32)
            for s in splits:
                j = jnp.where(s <= x, j + 1, j)
            for _lvl in range(GATHER_LEVELS):
                m = plsc.load_gather(t_v, [j])
                j = j + j + jnp.where(m <= x, 2, 1)
            o_v[pl.ds(i, L)] = plsc.load_gather(t_v, [j])

        out_copies.append(
            pltpu.async_copy(
                o_v.at[pl.ds(q * QC, QC)],
                out_hbm.at[pl.ds(base + q * QC, QC)],
                out_sem,
            )
        )

    for c in out_copies:
        c.wait()


@functools.cache
def _make_vq():
    cp = pltpu.CompilerParams()
    if "needs_layout_passes" in pltpu.CompilerParams.__dataclass_fields__:
        cp = dataclasses.replace(cp, needs_layout_passes=False)
    return pl.kernel(
        _vq_body,
        compiler_params=cp,
        out_type=jax.ShapeDtypeStruct((N,), jnp.float32),
        mesh=plsc.VectorSubcoreMesh(core_axis_name="c", subcore_axis_name="s"),
        scratch_types=[
            pltpu.VMEM((T_LEN,), jnp.float32),
            pltpu.VMEM((CHUNK,), jnp.float32),
            pltpu.VMEM((CHUNK,), jnp.float32),
            pltpu.SemaphoreType.DMA,
            pltpu.SemaphoreType.DMA,
            pltpu.SemaphoreType.DMA,
            pltpu.SemaphoreType.DMA,
            pltpu.SemaphoreType.DMA,
        ],
    )


@jax.jit
def kernel(param, y, codebook):
    cb_col = codebook.reshape(K, 1)
    cb_row = codebook.reshape(1, K)
    tree = _prep(cb_col, cb_row, jnp.asarray(_A_MAP), jnp.asarray(_B_MAP))
    quantized = _make_vq()(param, tree.reshape(T_LEN))
    return (quantized, y)
